# baseline (speedup 1.0000x reference)
import functools

import jax
import jax.numpy as jnp
from jax import lax
from jax.experimental import pallas as pl
from jax.experimental.pallas import tpu as pltpu
from jax.experimental.pallas import tpu_sc as plsc

_NC = 2
_NS = 16
_NW = _NC * _NS
_CH = 32
_SLOTS = 2


def _stream_slab(src_hbm, src_base, out_hbm, dst_base, stage, in_sems, out_sems, nch):
    in_copies = [
        pltpu.make_async_copy(
            src_hbm.at[pl.ds(src_base + j * _CH, _CH)],
            stage.at[j % _SLOTS],
            in_sems.at[j % _SLOTS],
        )
        for j in range(nch)
    ]
    out_copies = []
    for j in range(min(_SLOTS, nch)):
        in_copies[j].start()
    for j in range(nch):
        slot = j % _SLOTS
        in_copies[j].wait()
        oc = pltpu.make_async_copy(
            stage.at[slot],
            out_hbm.at[pl.ds(dst_base + j * _CH, _CH)],
            out_sems.at[slot],
        )
        oc.start()
        out_copies.append(oc)
        nxt = j + _SLOTS
        if nxt < nch:
            oc.wait()
            in_copies[nxt].start()
    for j in range(max(0, nch - _SLOTS), nch):
        out_copies[j].wait()


def kernel(inputs, buffer):
    b, d = inputs.shape
    n_steps = buffer.shape[0]
    seg = n_steps // b
    n_outer = n_steps // seg
    rows_w = n_outer // _NW
    nch = rows_w // _CH
    in_outer = b // seg
    n_in_workers = in_outer // rows_w

    mesh = plsc.VectorSubcoreMesh(core_axis_name="c", subcore_axis_name="s")

    @functools.partial(
        pl.kernel,
        out_type=jax.ShapeDtypeStruct((b, seg, d), inputs.dtype),
        mesh=mesh,
        scratch_types=[
            pltpu.MemorySpace.VMEM((_SLOTS, _CH, seg, d), jnp.float32),
            pltpu.SemaphoreType.DMA((_SLOTS,)),
            pltpu.SemaphoreType.DMA((_SLOTS,)),
        ],
    )
    def run(inputs_hbm, buffer_hbm, out_hbm, stage, in_sems, out_sems):
        inputs3 = inputs_hbm.reshape(b // seg, seg, d)
        buffer3 = buffer_hbm.reshape(n_outer, seg, d)
        cid = lax.axis_index("c")
        sid = lax.axis_index("s")
        wid = sid * _NC + cid
        base = wid * rows_w

        @pl.when(wid < n_in_workers)
        def _():
            _stream_slab(inputs3, base, out_hbm, base, stage, in_sems, out_sems, nch)

        @pl.when(wid >= n_in_workers)
        def _():
            _stream_slab(buffer3, base - in_outer, out_hbm, base, stage, in_sems, out_sems, nch)

    return run(inputs, buffer)


# speedup vs baseline: 1.0009x; 1.0009x over previous
import functools

import jax
import jax.numpy as jnp
from jax import lax
from jax.experimental import pallas as pl
from jax.experimental.pallas import tpu as pltpu
from jax.experimental.pallas import tpu_sc as plsc

_NC = 2
_NS = 16
_NW = _NC * _NS
_CH = 32
_SLOTS = 2


def _stream_slab(src_hbm, src_base, out_hbm, dst_base, stage, in_sems, out_sems, nch):
    in_copies = [
        pltpu.make_async_copy(
            src_hbm.at[pl.ds(src_base + j * _CH, _CH)],
            stage.at[j % _SLOTS],
            in_sems.at[j % _SLOTS],
        )
        for j in range(nch)
    ]
    out_copies = []
    for j in range(min(_SLOTS, nch)):
        in_copies[j].start()
    for j in range(nch):
        slot = j % _SLOTS
        in_copies[j].wait()
        oc = pltpu.make_async_copy(
            stage.at[slot],
            out_hbm.at[pl.ds(dst_base + j * _CH, _CH)],
            out_sems.at[slot],
        )
        oc.start()
        out_copies.append(oc)
        nxt = j + _SLOTS
        if nxt < nch:
            oc.wait()
            in_copies[nxt].start()
    for j in range(max(0, nch - _SLOTS), nch):
        out_copies[j].wait()


def kernel(inputs, buffer):
    b, d = inputs.shape
    n_steps = buffer.shape[0]
    seg = n_steps // b
    n_outer = n_steps // seg
    rows_w = n_outer // _NW
    nch = rows_w // _CH
    in_outer = b // seg
    n_in_workers = in_outer // rows_w

    mesh = plsc.VectorSubcoreMesh(core_axis_name="c", subcore_axis_name="s")

    @functools.partial(
        pl.kernel,
        out_type=jax.ShapeDtypeStruct((b, seg, d), inputs.dtype),
        mesh=mesh,
        scratch_types=[
            pltpu.MemorySpace.VMEM((_SLOTS, _CH, seg, d), jnp.float32),
            pltpu.SemaphoreType.DMA((_SLOTS,)),
            pltpu.SemaphoreType.DMA((_SLOTS,)),
        ],
        compiler_params=pltpu.CompilerParams(use_tc_tiling_on_sc=True),
    )
    def run(inputs_hbm, buffer_hbm, out_hbm, stage, in_sems, out_sems):
        inputs3 = inputs_hbm.reshape(b // seg, seg, d)
        buffer3 = buffer_hbm.reshape(n_outer, seg, d)
        cid = lax.axis_index("c")
        sid = lax.axis_index("s")
        wid = sid * _NC + cid
        base = wid * rows_w

        @pl.when(wid < n_in_workers)
        def _():
            _stream_slab(inputs3, base, out_hbm, base, stage, in_sems, out_sems, nch)

        @pl.when(wid >= n_in_workers)
        def _():
            _stream_slab(buffer3, base - in_outer, out_hbm, base, stage, in_sems, out_sems, nch)

    return run(inputs, buffer)


# speedup vs baseline: 1.1836x; 1.1826x over previous
import jax
import jax.numpy as jnp
from jax.experimental import pallas as pl


def _copy_body(inputs_ref, buffer_ref, out_ref):
    i = pl.program_id(0)
    bo, seg, d = out_ref.shape

    @pl.when(i == 0)
    def _():
        out_ref[...] = inputs_ref[...].reshape(bo, seg, d)

    @pl.when(i > 0)
    def _():
        out_ref[...] = buffer_ref[...].reshape(bo, seg, d)


def kernel(inputs, buffer):
    b, d = inputs.shape
    n_steps = buffer.shape[0]
    seg = n_steps // b
    n_blocks = n_steps // b
    bo = b // seg

    return pl.pallas_call(
        _copy_body,
        grid=(n_blocks,),
        in_specs=[
            pl.BlockSpec((b, d), lambda i: (0, 0)),
            pl.BlockSpec((b, d), lambda i: (jnp.maximum(i - 1, 0), 0)),
        ],
        out_specs=pl.BlockSpec((bo, seg, d), lambda i: (i, 0, 0)),
        out_shape=jax.ShapeDtypeStruct((b, seg, d), inputs.dtype),
    )(inputs, buffer)
